# baseline (speedup 1.0000x reference)
import functools

import jax
import jax.numpy as jnp
from jax import lax
from jax.experimental import pallas as pl
from jax.experimental.pallas import tpu as pltpu
from jax.experimental.pallas import tpu_sc as plsc

N = 50000
E = 150000
P = 50000
DF = 128
DH = 256
R = 3

NBLK = 49
BLK = 1024
NROW = NBLK * BLK
NPAD = 61440

EPAD = 150016
EPT = EPAD // 16
NG = EPT // 16

TOTE = 3 * E + P
TOTP = 524288
EPT_E = TOTP // 32
EB = 1024
NB_E = EPT_E // EB

_MESH = plsc.VectorSubcoreMesh(core_axis_name="c", subcore_axis_name="s")
_SC_PARAMS = pltpu.CompilerParams(needs_layout_passes=False)


def _make_segsum(DW, CS, FBW, with_deg):
    NCH = NPAD // CS
    RPT = CS // 16
    RPTW = RPT * DW
    FBN = FBW // DW

    BB = 32

    outs = [jax.ShapeDtypeStruct((R, NPAD * DW, 128), jnp.float32)]
    scratch = [
        pltpu.VMEM_SHARED(((CS + 16) * DW, 128), jnp.float32),
        pltpu.VMEM((EPT,), jnp.int32),
        pltpu.VMEM((EPT,), jnp.int32),
        pltpu.VMEM((EPT + 48,), jnp.int32),
        pltpu.VMEM((EPT + 48,), jnp.int32),
        pltpu.VMEM((BB * DW, 128), jnp.float32),
        pltpu.VMEM((BB * DW, 128), jnp.float32),
        pltpu.VMEM((BB * DW,), jnp.int32),
        pltpu.VMEM((BB * DW,), jnp.int32),
        pltpu.VMEM((BB * DW,), jnp.int32),
        pltpu.VMEM((BB * DW,), jnp.int32),
        pltpu.VMEM((FBW, 128), jnp.float32),
        pltpu.SemaphoreType.DMA,
        pltpu.SemaphoreType.DMA,
    ]
    if with_deg:
        outs.append(jax.ShapeDtypeStruct((R * NPAD,), jnp.float32))
        scratch.insert(-2, pltpu.VMEM((CS + 16,), jnp.float32))
        scratch.insert(-2, pltpu.VMEM_SHARED((16 * CS,), jnp.float32))
        scratch.insert(-2, pltpu.VMEM((RPT,), jnp.float32))
        scratch.insert(-2, pltpu.VMEM((RPT,), jnp.float32))

    def seg_body(refs):
        if with_deg:
            (tab_hbm, src_hbm, dst_hbm, zeros_hbm, out_hbm, deg_out, acc,
             dstv, srcv, ssrc, soff, rows0, rows1, gi0, gi1, si0, si1, fbuf,
             degp, sdeg, tmpd, degv, sem0, sem1) = refs
        else:
            (tab_hbm, src_hbm, dst_hbm, zeros_hbm, out_hbm, acc,
             dstv, srcv, ssrc, soff, rows0, rows1, gi0, gi1, si0, si1, fbuf,
             sem0, sem1) = refs
        bufs = [rows0, rows1]
        gidx = [gi0, gi1]
        sidx = [si0, si1]
        sems = [sem0, sem1]
        core = lax.axis_index("c")
        sid = lax.axis_index("s")
        ones = jnp.ones((16,), jnp.float32)

        for r in range(R):
            pltpu.sync_copy(dst_hbm.at[pl.ds(r * EPAD + sid * EPT, EPT)],
                            dstv)
            pltpu.sync_copy(src_hbm.at[pl.ds(r * EPAD + sid * EPT, EPT)],
                            srcv)

            def chunk_body(j, _):
                lo = (2 * j + core) * CS
                @pl.when(lo < N)
                def _chunk():
                    _do_chunk(lo)
                return 0

            def _do_chunk(lo):
                plsc.subcore_barrier()

                def zacc(q, _):
                    pltpu.sync_copy(
                        zeros_hbm,
                        acc.at[pl.ds(sid * RPTW + q * FBW, FBW)])
                    return 0

                lax.fori_loop(0, RPTW // FBW, zacc, 0)
                if with_deg:
                    def zd(g, _):
                        degp[pl.ds(g * 16, 16)] = jnp.zeros((16,),
                                                            jnp.float32)
                        return 0
                    lax.fori_loop(0, (CS + 16) // 16, zd, 0)
                plsc.subcore_barrier()

                def scan_body(g, cnt):
                    base = g * 16
                    voff = dstv[pl.ds(base, 16)] - lo
                    m = (voff >= 0) & (voff < CS)
                    vsrc = srcv[pl.ds(base, 16)]
                    mi = m.astype(jnp.int32)
                    dest = plsc.cumsum(mi) + (cnt - 1)
                    plsc.store_scatter(ssrc, [dest], vsrc, mask=m)
                    plsc.store_scatter(soff, [dest], voff, mask=m)
                    if with_deg:
                        vcl = jnp.where(m, voff, CS)
                        plsc.addupdate_scatter(degp, [vcl], ones, mask=m)
                    return cnt + jnp.sum(mi)

                cnt = lax.fori_loop(0, NG, scan_body, 0)
                for t in range(BB // 16):
                    ssrc[pl.ds(cnt + t * 16, 16)] = jnp.zeros((16,),
                                                              jnp.int32)
                    soff[pl.ds(cnt + t * 16, 16)] = jnp.full((16,), CS,
                                                             jnp.int32)

                nb = (cnt + BB - 1) // BB
                it16 = lax.iota(jnp.int32, 16)

                def fire(b, k):
                    if DW == 1:
                        pltpu.async_copy(
                            tab_hbm.at[ssrc.at[pl.ds(b * BB, BB)]], bufs[k],
                            sems[k])
                    else:
                        for g16 in range(BB // 16):
                            vs = ssrc[pl.ds(b * BB + g16 * 16, 16)]
                            vo = soff[pl.ds(b * BB + g16 * 16, 16)]
                            dst0 = 2 * it16 + g16 * 32
                            plsc.store_scatter(gidx[k], [dst0], 2 * vs)
                            plsc.store_scatter(gidx[k], [dst0 + 1],
                                               2 * vs + 1)
                            plsc.store_scatter(sidx[k], [dst0], 2 * vo)
                            plsc.store_scatter(sidx[k], [dst0 + 1],
                                               2 * vo + 1)
                        pltpu.async_copy(tab_hbm.at[gidx[k]], bufs[k],
                                         sems[k])

                for i in range(2):
                    @pl.when(i < nb)
                    def _(i=i):
                        fire(i, i)

                def gs2(h, _):
                    for k in range(2):
                        b = h * 2 + k

                        @pl.when(b < nb)
                        def _(b=b, k=k):
                            if DW == 1:
                                pltpu.make_async_copy(
                                    tab_hbm.at[ssrc.at[pl.ds(b * BB, BB)]],
                                    bufs[k], sems[k]).wait()
                            else:
                                pltpu.make_async_copy(
                                    tab_hbm.at[gidx[k]], bufs[k],
                                    sems[k]).wait()
                            for k16 in range(BB * DW // 16):
                                if DW == 1:
                                    vo = soff[pl.ds(b * BB + k16 * 16, 16)]
                                else:
                                    vo = sidx[k][pl.ds(k16 * 16, 16)]
                                pltpu.sync_copy(
                                    bufs[k].at[pl.ds(k16 * 16, 16)],
                                    acc.at[vo], add=True)

                            @pl.when(b + 2 < nb)
                            def _(b=b, k=k):
                                fire(b + 2, k)
                    return 0

                lax.fori_loop(0, (nb + 1) // 2, gs2, 0)
                if with_deg:
                    pltpu.sync_copy(degp.at[pl.ds(0, CS)],
                                    sdeg.at[pl.ds(sid * CS, CS)])
                plsc.subcore_barrier()
                if with_deg:
                    pltpu.sync_copy(sdeg.at[pl.ds(sid * RPT, RPT)], degv)

                    def red_body(k, _):
                        pltpu.sync_copy(
                            sdeg.at[pl.ds(k * CS + sid * RPT, RPT)], tmpd)

                        def addb(g, _):
                            sl = pl.ds(g * 16, 16)
                            degv[sl] = degv[sl] + tmpd[sl]
                            return 0

                        lax.fori_loop(0, RPT // 16, addb, 0)
                        return 0

                    lax.fori_loop(1, 16, red_body, 0)
                    pltpu.sync_copy(
                        degv,
                        deg_out.at[pl.ds(r * NPAD + lo + sid * RPT, RPT)])

                def flush_body(q, _):
                    row0w = sid * RPTW + q * FBW
                    pltpu.sync_copy(
                        acc.at[pl.ds(row0w, FBW)],
                        out_hbm.at[r, pl.ds(lo * DW + row0w, FBW), :])
                    return 0

                lax.fori_loop(0, RPTW // FBW, flush_body, 0)

            lax.fori_loop(0, NCH // 2, chunk_body, 0)

    if with_deg:
        def seg(tab_hbm, src_hbm, dst_hbm, zeros_hbm, out_hbm, deg_out, *sc):
            seg_body((tab_hbm, src_hbm, dst_hbm, zeros_hbm, out_hbm,
                      deg_out) + sc)
    else:
        def seg(tab_hbm, src_hbm, dst_hbm, zeros_hbm, out_hbm, *sc):
            seg_body((tab_hbm, src_hbm, dst_hbm, zeros_hbm, out_hbm) + sc)

    return pl.kernel(
        seg,
        mesh=_MESH,
        out_type=outs if with_deg else outs[0],
        scratch_types=scratch,
        compiler_params=_SC_PARAMS,
    )


_segsum1 = _make_segsum(1, 7680, 32, True)
_segsum2 = _make_segsum(2, 3840, 32, False)


@functools.partial(
    pl.kernel,
    mesh=_MESH,
    out_type=jax.ShapeDtypeStruct((TOTP,), jnp.float32),
    scratch_types=[
        pltpu.VMEM((N,), jnp.float32),
        pltpu.VMEM((N,), jnp.float32),
        pltpu.VMEM((EB,), jnp.int32),
        pltpu.VMEM((EB,), jnp.int32),
        pltpu.VMEM((EB,), jnp.float32),
        pltpu.SemaphoreType.DMA,
    ],
    compiler_params=_SC_PARAMS,
)
def _edge_scores(u_hbm, v_hbm, a_hbm, b_hbm, out_hbm, uv, vv, av, bv, ov,
                 sem):
    core = lax.axis_index("c")
    sid = lax.axis_index("s")
    wid = sid * 2 + core
    pltpu.sync_copy(u_hbm, uv)
    pltpu.sync_copy(v_hbm, vv)
    base_t = wid * EPT_E
    for bi in range(NB_E):
        off = base_t + bi * EB
        pltpu.sync_copy(a_hbm.at[pl.ds(off, EB)], av)
        pltpu.sync_copy(b_hbm.at[pl.ds(off, EB)], bv)

        def body(g, _):
            su = plsc.load_gather(uv, [av[pl.ds(g * 16, 16)]])
            sv = plsc.load_gather(vv, [bv[pl.ds(g * 16, 16)]])
            s = su + sv
            ov[pl.ds(g * 16, 16)] = 1.0 / (1.0 + jnp.exp(-s))
            return 0

        lax.fori_loop(0, EB // 16, body, 0)
        pltpu.sync_copy(ov, out_hbm.at[pl.ds(off, EB)])


def _layer1_body(x_ref, agg_ref, deg_ref, w_ref, b_ref, o_ref):
    parts = [x_ref[...]]
    for r in range(R):
        rd = 1.0 / jnp.maximum(deg_ref[r, :], 1.0)
        parts.append(agg_ref[r, :, :] * rd.reshape(BLK, 1))
    cat = jnp.concatenate(parts, axis=1)
    h = jnp.dot(cat, w_ref[...], preferred_element_type=jnp.float32)
    o_ref[...] = jnp.maximum(h + b_ref[...], 0.0)


def _layer2_body(h1_ref, agg2_ref, deg_ref, w_ref, b_ref, wuv_ref, buv_ref,
                 o_ref):
    parts = [h1_ref[...]]
    for r in range(R):
        rd = 1.0 / jnp.maximum(deg_ref[r, :], 1.0)
        parts.append(agg2_ref[r, :, :] * rd.reshape(BLK, 1))
    cat = jnp.concatenate(parts, axis=1)
    g = jnp.maximum(
        jnp.dot(cat, w_ref[...], preferred_element_type=jnp.float32)
        + b_ref[...], 0.0)
    o_ref[...] = (jnp.dot(g, wuv_ref[...], preferred_element_type=jnp.float32)
                  + buv_ref[...])


def _layer1(x, agg1, deg2d, w1cat, b1bar):
    return pl.pallas_call(
        _layer1_body,
        grid=(NBLK,),
        in_specs=[
            pl.BlockSpec((BLK, DF), lambda i: (i, 0)),
            pl.BlockSpec((R, BLK, DF), lambda i: (0, i, 0)),
            pl.BlockSpec((R, BLK), lambda i: (0, i)),
            pl.BlockSpec((4 * DF, DH), lambda i: (0, 0)),
            pl.BlockSpec((1, DH), lambda i: (0, 0)),
        ],
        out_specs=pl.BlockSpec((BLK, DH), lambda i: (i, 0)),
        out_shape=jax.ShapeDtypeStruct((NROW, DH), jnp.float32),
    )(x, agg1, deg2d, w1cat, b1bar)


def _layer2(h1, agg2, deg2d, w2cat, b2bar, wuv, buv):
    return pl.pallas_call(
        _layer2_body,
        grid=(NBLK,),
        in_specs=[
            pl.BlockSpec((BLK, DH), lambda i: (i, 0)),
            pl.BlockSpec((R, BLK, DH), lambda i: (0, i, 0)),
            pl.BlockSpec((R, BLK), lambda i: (0, i)),
            pl.BlockSpec((4 * DH, DF), lambda i: (0, 0)),
            pl.BlockSpec((1, DF), lambda i: (0, 0)),
            pl.BlockSpec((DF, DF), lambda i: (0, 0)),
            pl.BlockSpec((1, DF), lambda i: (0, 0)),
        ],
        out_specs=pl.BlockSpec((BLK, DF), lambda i: (i, 0)),
        out_shape=jax.ShapeDtypeStruct((NROW, DF), jnp.float32),
    )(h1, agg2, deg2d, w2cat, b2bar, wuv, buv)


def kernel(x, edge_index_0, edge_index_1, edge_index_2, n_pairs, W1_self,
           W1_neigh, b1, W2_self, W2_neigh, b2, Wlin, blin):
    edges = [edge_index_0, edge_index_1, edge_index_2]

    w1cat = jnp.concatenate(
        [jnp.mean(W1_self, 0)] + [W1_neigh[r] / R for r in range(R)], axis=0)
    b1bar = jnp.mean(b1, 0)[None, :]
    w2cat = jnp.concatenate(
        [jnp.mean(W2_self, 0)] + [W2_neigh[r] / R for r in range(R)], axis=0)
    b2bar = jnp.mean(b2, 0)[None, :]
    wuv = jnp.zeros((DF, DF), jnp.float32)
    wuv = wuv.at[:, 0].set(Wlin[:DF, 0]).at[:, 1].set(Wlin[DF:, 0])
    buv = jnp.zeros((1, DF), jnp.float32).at[0, 0].set(blin[0])

    ed = jnp.pad(jnp.stack(edges), ((0, 0), (0, 0), (0, EPAD - E)),
                 constant_values=NPAD)
    src_flat = ed[:, 0, :].reshape(-1)
    dst_flat = ed[:, 1, :].reshape(-1)
    zeros = jnp.zeros((32, 128), jnp.float32)

    x_pad = jnp.pad(x, ((0, NROW - N), (0, 0)))
    agg1, deg_flat = _segsum1(x, src_flat, dst_flat, zeros)
    deg2d = deg_flat.reshape(R, NPAD)
    h1 = _layer1(x_pad, agg1, deg2d, w1cat, b1bar)
    agg2w = _segsum2(h1.reshape(2 * NROW, 128), src_flat, dst_flat, zeros)
    agg2 = agg2w.reshape(R, NPAD, DH)
    uv = _layer2(h1, agg2, deg2d, w2cat, b2bar, wuv, buv)

    a_list = jnp.concatenate(
        [e[0] for e in edges] + [n_pairs[:, 0],
                                 jnp.zeros((TOTP - TOTE,), jnp.int32)])
    b_list = jnp.concatenate(
        [e[1] for e in edges] + [n_pairs[:, 1],
                                 jnp.zeros((TOTP - TOTE,), jnp.int32)])
    scores = _edge_scores(uv[:N, 0], uv[:N, 1], a_list, b_list)
    return scores[:TOTE, None]


# speedup vs baseline: 1.2743x; 1.2743x over previous
import functools

import jax
import jax.numpy as jnp
from jax import lax
from jax.experimental import pallas as pl
from jax.experimental.pallas import tpu as pltpu
from jax.experimental.pallas import tpu_sc as plsc

N = 50000
E = 150000
P = 50000
DF = 128
DH = 256
R = 3

NBLK = 49
BLK = 1024
NROW = NBLK * BLK
NPAD = 61440

EPAD = 150016
EPT = EPAD // 16
NG = EPT // 16

TOTE = 3 * E + P
TOTP = 524288
EPT_E = TOTP // 32
EB = 1024
NB_E = EPT_E // EB

_MESH = plsc.VectorSubcoreMesh(core_axis_name="c", subcore_axis_name="s")
_SC_PARAMS = pltpu.CompilerParams(needs_layout_passes=False)


def _make_segsum(DW, CS, FBW, with_deg, NPD):
    NCH = NPD // CS
    RPT = CS // 16
    RPTW = RPT * DW
    FBN = FBW // DW

    BB = 32

    outs = [jax.ShapeDtypeStruct((R, NPD * DW, 128), jnp.float32)]
    scratch = [
        pltpu.VMEM_SHARED(((CS + 16) * DW, 128), jnp.float32),
        pltpu.VMEM((EPT,), jnp.int32),
        pltpu.VMEM((EPT,), jnp.int32),
        pltpu.VMEM((EPT + 48,), jnp.int32),
        pltpu.VMEM((EPT + 48,), jnp.int32),
        pltpu.VMEM((BB * DW, 128), jnp.float32),
        pltpu.VMEM((BB * DW, 128), jnp.float32),
        pltpu.VMEM((BB * DW,), jnp.int32),
        pltpu.VMEM((BB * DW,), jnp.int32),
        pltpu.VMEM((BB * DW,), jnp.int32),
        pltpu.VMEM((BB * DW,), jnp.int32),
        pltpu.VMEM((FBW, 128), jnp.float32),
        pltpu.SemaphoreType.DMA,
        pltpu.SemaphoreType.DMA,
    ]
    if with_deg:
        outs.append(jax.ShapeDtypeStruct((R * NPD,), jnp.float32))
        scratch.insert(-2, pltpu.VMEM((CS + 16,), jnp.float32))
        scratch.insert(-2, pltpu.VMEM_SHARED((16 * CS,), jnp.float32))
        scratch.insert(-2, pltpu.VMEM((RPT,), jnp.float32))
        scratch.insert(-2, pltpu.VMEM((RPT,), jnp.float32))

    def seg_body(refs):
        if with_deg:
            (tab_hbm, src_hbm, dst_hbm, zeros_hbm, out_hbm, deg_out, acc,
             dstv, srcv, ssrc, soff, rows0, rows1, gi0, gi1, si0, si1, fbuf,
             degp, sdeg, tmpd, degv, sem0, sem1) = refs
        else:
            (tab_hbm, src_hbm, dst_hbm, zeros_hbm, out_hbm, acc,
             dstv, srcv, ssrc, soff, rows0, rows1, gi0, gi1, si0, si1, fbuf,
             sem0, sem1) = refs
        bufs = [rows0, rows1]
        gidx = [gi0, gi1]
        sidx = [si0, si1]
        sems = [sem0, sem1]
        core = lax.axis_index("c")
        sid = lax.axis_index("s")
        ones = jnp.ones((16,), jnp.float32)

        for r in range(R):
            pltpu.sync_copy(dst_hbm.at[pl.ds(r * EPAD + sid * EPT, EPT)],
                            dstv)
            pltpu.sync_copy(src_hbm.at[pl.ds(r * EPAD + sid * EPT, EPT)],
                            srcv)

            def chunk_body(j, _):
                lo = (2 * j + core) * CS
                @pl.when(lo < N)
                def _chunk():
                    _do_chunk(lo)
                return 0

            def _do_chunk(lo):
                plsc.subcore_barrier()
                pltpu.sync_copy(zeros_hbm, fbuf)

                def zacc(q, _):
                    pltpu.sync_copy(
                        fbuf, acc.at[pl.ds(sid * RPTW + q * FBW, FBW)])
                    return 0

                lax.fori_loop(0, RPTW // FBW, zacc, 0)
                if with_deg:
                    def zd(g, _):
                        degp[pl.ds(g * 16, 16)] = jnp.zeros((16,),
                                                            jnp.float32)
                        return 0
                    lax.fori_loop(0, (CS + 16) // 16, zd, 0)
                plsc.subcore_barrier()

                def scan_body(g, cnt):
                    base = g * 16
                    voff = dstv[pl.ds(base, 16)] - lo
                    m = (voff >= 0) & (voff < CS)
                    vsrc = srcv[pl.ds(base, 16)]
                    mi = m.astype(jnp.int32)
                    dest = plsc.cumsum(mi) + (cnt - 1)
                    plsc.store_scatter(ssrc, [dest], vsrc, mask=m)
                    plsc.store_scatter(soff, [dest], voff, mask=m)
                    if with_deg:
                        vcl = jnp.where(m, voff, CS)
                        plsc.addupdate_scatter(degp, [vcl], ones, mask=m)
                    return cnt + jnp.sum(mi)

                cnt = lax.fori_loop(0, NG, scan_body, 0)
                for t in range(BB // 16):
                    ssrc[pl.ds(cnt + t * 16, 16)] = jnp.zeros((16,),
                                                              jnp.int32)
                    soff[pl.ds(cnt + t * 16, 16)] = jnp.full((16,), CS,
                                                             jnp.int32)

                nb = (cnt + BB - 1) // BB
                it16 = lax.iota(jnp.int32, 16)

                def fire(b, k):
                    if DW == 1:
                        pltpu.async_copy(
                            tab_hbm.at[ssrc.at[pl.ds(b * BB, BB)]], bufs[k],
                            sems[k])
                    else:
                        for g16 in range(BB // 16):
                            vs = ssrc[pl.ds(b * BB + g16 * 16, 16)]
                            vo = soff[pl.ds(b * BB + g16 * 16, 16)]
                            dst0 = 2 * it16 + g16 * 32
                            plsc.store_scatter(gidx[k], [dst0], 2 * vs)
                            plsc.store_scatter(gidx[k], [dst0 + 1],
                                               2 * vs + 1)
                            plsc.store_scatter(sidx[k], [dst0], 2 * vo)
                            plsc.store_scatter(sidx[k], [dst0 + 1],
                                               2 * vo + 1)
                        pltpu.async_copy(tab_hbm.at[gidx[k]], bufs[k],
                                         sems[k])

                for i in range(2):
                    @pl.when(i < nb)
                    def _(i=i):
                        fire(i, i)

                def gs2(h, _):
                    for k in range(2):
                        b = h * 2 + k

                        @pl.when(b < nb)
                        def _(b=b, k=k):
                            if DW == 1:
                                pltpu.make_async_copy(
                                    tab_hbm.at[ssrc.at[pl.ds(b * BB, BB)]],
                                    bufs[k], sems[k]).wait()
                            else:
                                pltpu.make_async_copy(
                                    tab_hbm.at[gidx[k]], bufs[k],
                                    sems[k]).wait()
                            for k16 in range(BB * DW // 16):
                                if DW == 1:
                                    vo = soff[pl.ds(b * BB + k16 * 16, 16)]
                                else:
                                    vo = sidx[k][pl.ds(k16 * 16, 16)]
                                pltpu.sync_copy(
                                    bufs[k].at[pl.ds(k16 * 16, 16)],
                                    acc.at[vo], add=True)

                            @pl.when(b + 2 < nb)
                            def _(b=b, k=k):
                                fire(b + 2, k)
                    return 0

                lax.fori_loop(0, (nb + 1) // 2, gs2, 0)
                if with_deg:
                    pltpu.sync_copy(degp.at[pl.ds(0, CS)],
                                    sdeg.at[pl.ds(sid * CS, CS)])
                plsc.subcore_barrier()
                if with_deg:
                    pltpu.sync_copy(sdeg.at[pl.ds(sid * RPT, RPT)], degv)

                    def red_body(k, _):
                        pltpu.sync_copy(
                            sdeg.at[pl.ds(k * CS + sid * RPT, RPT)], tmpd)

                        def addb(g, _):
                            sl = pl.ds(g * 16, 16)
                            degv[sl] = degv[sl] + tmpd[sl]
                            return 0

                        lax.fori_loop(0, RPT // 16, addb, 0)
                        return 0

                    lax.fori_loop(1, 16, red_body, 0)
                    pltpu.sync_copy(
                        degv,
                        deg_out.at[pl.ds(r * NPD + lo + sid * RPT, RPT)])

                def flush_body(q, _):
                    row0w = sid * RPTW + q * FBW
                    pltpu.sync_copy(acc.at[pl.ds(row0w, FBW)], fbuf)
                    pltpu.sync_copy(
                        fbuf,
                        out_hbm.at[r, pl.ds(lo * DW + row0w, FBW), :])
                    return 0

                lax.fori_loop(0, RPTW // FBW, flush_body, 0)

            lax.fori_loop(0, NCH // 2, chunk_body, 0)

    if with_deg:
        def seg(tab_hbm, src_hbm, dst_hbm, zeros_hbm, out_hbm, deg_out, *sc):
            seg_body((tab_hbm, src_hbm, dst_hbm, zeros_hbm, out_hbm,
                      deg_out) + sc)
    else:
        def seg(tab_hbm, src_hbm, dst_hbm, zeros_hbm, out_hbm, *sc):
            seg_body((tab_hbm, src_hbm, dst_hbm, zeros_hbm, out_hbm) + sc)

    return pl.kernel(
        seg,
        mesh=_MESH,
        out_type=outs if with_deg else outs[0],
        scratch_types=scratch,
        compiler_params=_SC_PARAMS,
    )


NPAD2 = 52224
_segsum1 = _make_segsum(1, 7680, 32, True, NPAD)
_segsum2 = _make_segsum(2, 4352, 32, False, NPAD2)


@functools.partial(
    pl.kernel,
    mesh=_MESH,
    out_type=jax.ShapeDtypeStruct((TOTP,), jnp.float32),
    scratch_types=[
        pltpu.VMEM((N,), jnp.float32),
        pltpu.VMEM((N,), jnp.float32),
        pltpu.VMEM((EB,), jnp.int32),
        pltpu.VMEM((EB,), jnp.int32),
        pltpu.VMEM((EB,), jnp.float32),
        pltpu.SemaphoreType.DMA,
    ],
    compiler_params=_SC_PARAMS,
)
def _edge_scores(u_hbm, v_hbm, a_hbm, b_hbm, out_hbm, uv, vv, av, bv, ov,
                 sem):
    core = lax.axis_index("c")
    sid = lax.axis_index("s")
    wid = sid * 2 + core
    pltpu.sync_copy(u_hbm, uv)
    pltpu.sync_copy(v_hbm, vv)
    base_t = wid * EPT_E
    for bi in range(NB_E):
        off = base_t + bi * EB
        pltpu.sync_copy(a_hbm.at[pl.ds(off, EB)], av)
        pltpu.sync_copy(b_hbm.at[pl.ds(off, EB)], bv)

        def body(g, _):
            su = plsc.load_gather(uv, [av[pl.ds(g * 16, 16)]])
            sv = plsc.load_gather(vv, [bv[pl.ds(g * 16, 16)]])
            s = su + sv
            ov[pl.ds(g * 16, 16)] = 1.0 / (1.0 + jnp.exp(-s))
            return 0

        lax.fori_loop(0, EB // 16, body, 0)
        pltpu.sync_copy(ov, out_hbm.at[pl.ds(off, EB)])


def _layer1_body(x_ref, agg_ref, deg_ref, w_ref, b_ref, o_ref):
    parts = [x_ref[...]]
    for r in range(R):
        rd = 1.0 / jnp.maximum(deg_ref[r, :], 1.0)
        parts.append(agg_ref[r, :, :] * rd.reshape(BLK, 1))
    cat = jnp.concatenate(parts, axis=1)
    h = jnp.dot(cat, w_ref[...], preferred_element_type=jnp.float32)
    o_ref[...] = jnp.maximum(h + b_ref[...], 0.0)


def _layer2_body(h1_ref, agg2_ref, deg_ref, w_ref, b_ref, wuv_ref, buv_ref,
                 o_ref):
    parts = [h1_ref[...]]
    for r in range(R):
        rd = 1.0 / jnp.maximum(deg_ref[r, :], 1.0)
        parts.append(agg2_ref[r, :, :] * rd.reshape(BLK, 1))
    cat = jnp.concatenate(parts, axis=1)
    g = jnp.maximum(
        jnp.dot(cat, w_ref[...], preferred_element_type=jnp.float32)
        + b_ref[...], 0.0)
    o_ref[...] = (jnp.dot(g, wuv_ref[...], preferred_element_type=jnp.float32)
                  + buv_ref[...])


def _layer1(x, agg1, deg2d, w1cat, b1bar):
    return pl.pallas_call(
        _layer1_body,
        grid=(NBLK,),
        in_specs=[
            pl.BlockSpec((BLK, DF), lambda i: (i, 0)),
            pl.BlockSpec((R, BLK, DF), lambda i: (0, i, 0)),
            pl.BlockSpec((R, BLK), lambda i: (0, i)),
            pl.BlockSpec((4 * DF, DH), lambda i: (0, 0)),
            pl.BlockSpec((1, DH), lambda i: (0, 0)),
        ],
        out_specs=pl.BlockSpec((BLK, DH), lambda i: (i, 0)),
        out_shape=jax.ShapeDtypeStruct((NROW, DH), jnp.float32),
    )(x, agg1, deg2d, w1cat, b1bar)


def _layer2(h1, agg2, deg2d, w2cat, b2bar, wuv, buv):
    return pl.pallas_call(
        _layer2_body,
        grid=(NBLK,),
        in_specs=[
            pl.BlockSpec((BLK, DH), lambda i: (i, 0)),
            pl.BlockSpec((R, BLK, DH), lambda i: (0, i, 0)),
            pl.BlockSpec((R, BLK), lambda i: (0, i)),
            pl.BlockSpec((4 * DH, DF), lambda i: (0, 0)),
            pl.BlockSpec((1, DF), lambda i: (0, 0)),
            pl.BlockSpec((DF, DF), lambda i: (0, 0)),
            pl.BlockSpec((1, DF), lambda i: (0, 0)),
        ],
        out_specs=pl.BlockSpec((BLK, DF), lambda i: (i, 0)),
        out_shape=jax.ShapeDtypeStruct((NROW, DF), jnp.float32),
    )(h1, agg2, deg2d, w2cat, b2bar, wuv, buv)


def kernel(x, edge_index_0, edge_index_1, edge_index_2, n_pairs, W1_self,
           W1_neigh, b1, W2_self, W2_neigh, b2, Wlin, blin):
    edges = [edge_index_0, edge_index_1, edge_index_2]

    w1cat = jnp.concatenate(
        [jnp.mean(W1_self, 0)] + [W1_neigh[r] / R for r in range(R)], axis=0)
    b1bar = jnp.mean(b1, 0)[None, :]
    w2cat = jnp.concatenate(
        [jnp.mean(W2_self, 0)] + [W2_neigh[r] / R for r in range(R)], axis=0)
    b2bar = jnp.mean(b2, 0)[None, :]
    wuv = jnp.zeros((DF, DF), jnp.float32)
    wuv = wuv.at[:, 0].set(Wlin[:DF, 0]).at[:, 1].set(Wlin[DF:, 0])
    buv = jnp.zeros((1, DF), jnp.float32).at[0, 0].set(blin[0])

    ed = jnp.pad(jnp.stack(edges), ((0, 0), (0, 0), (0, EPAD - E)),
                 constant_values=NPAD)
    src_flat = ed[:, 0, :].reshape(-1)
    dst_flat = ed[:, 1, :].reshape(-1)
    zeros = jnp.zeros((32, 128), jnp.float32)

    x_pad = jnp.pad(x, ((0, NROW - N), (0, 0)))
    agg1, deg_flat = _segsum1(x, src_flat, dst_flat, zeros)
    deg2d = deg_flat.reshape(R, NPAD)
    h1 = _layer1(x_pad, agg1, deg2d, w1cat, b1bar)
    agg2w = _segsum2(h1.reshape(2 * NROW, 128), src_flat, dst_flat, zeros)
    agg2 = agg2w.reshape(R, NPAD2, DH)
    uv = _layer2(h1, agg2, deg2d, w2cat, b2bar, wuv, buv)

    a_list = jnp.concatenate(
        [e[0] for e in edges] + [n_pairs[:, 0],
                                 jnp.zeros((TOTP - TOTE,), jnp.int32)])
    b_list = jnp.concatenate(
        [e[1] for e in edges] + [n_pairs[:, 1],
                                 jnp.zeros((TOTP - TOTE,), jnp.int32)])
    scores = _edge_scores(uv[:N, 0], uv[:N, 1], a_list, b_list)
    return scores[:TOTE, None]
